# Initial kernel scaffold; baseline (speedup 1.0000x reference)
import jax
import jax.numpy as jnp
from jax.experimental import pallas as pl


def kernel(x, edge_index, batch, W1, b1, g1, bt1, W2, b2, g2, bt2, W3, b3, g3, bt3, fW1, fb1, fW2, fb2, fW3, fb3):
    raise NotImplementedError("write your pallas kernel here")


# speedup vs baseline: 14.7234x; 14.7234x over previous
import functools

import jax
import jax.numpy as jnp
from jax import lax
from jax.experimental import pallas as pl
from jax.experimental.pallas import tpu as pltpu
from jax.experimental.pallas import tpu_sc as plsc

N = 50000
E = 800000
B = 512
EPS = 1e-5

BN_ROWS = 1000
NB = N // BN_ROWS
EROWS = 6272
EPAD = EROWS * 128
NACC = N + 48
CH = 8


def _sc_mesh():
    return plsc.VectorSubcoreMesh(core_axis_name="c", subcore_axis_name="s")


_SC_PARAMS = pltpu.CompilerParams(use_tc_tiling_on_sc=False,
                                  needs_layout_passes=False)


def _make_deg_kernel():
    nchunk = (EROWS // 16) // CH

    @functools.partial(
        pl.kernel,
        mesh=_sc_mesh(),
        compiler_params=_SC_PARAMS,
        out_type=jax.ShapeDtypeStruct((NACC, 1), jnp.float32),
        scratch_types=[
            pltpu.VMEM((CH, 128), jnp.int32),
            pltpu.VMEM((128, 1), jnp.float32),
            pltpu.VMEM_SHARED((NACC, 1), jnp.float32),
            pltpu.SemaphoreType.DMA,
        ],
    )
    def k(dst_hbm, ones_hbm, zeros_hbm, out_hbm, dst_v, ones_v, acc_sh, sem):
        cid = lax.axis_index("c")
        sid = lax.axis_index("s")
        pltpu.sync_copy(ones_hbm, ones_v)
        base = sid * (NACC // 16)
        pltpu.sync_copy(zeros_hbm.at[pl.ds(0, NACC // 16)],
                        acc_sh.at[pl.ds(base, NACC // 16)])
        plsc.subcore_barrier()

        def chunk(i, _):
            rb = sid * (EROWS // 16) + i * CH
            pltpu.sync_copy(dst_hbm.at[pl.ds(rb, CH)], dst_v)
            cps = [
                pltpu.async_copy(ones_v, acc_sh.at[dst_v.at[r]], sem,
                                 add=True)
                for r in range(CH)
            ]
            for cp in cps:
                cp.wait()
            return 0

        lax.fori_loop(0, nchunk, chunk, 0)
        plsc.subcore_barrier()

        @pl.when((sid // 8) == cid)
        def _():
            wb = sid * (NACC // 16)
            pltpu.sync_copy(acc_sh.at[pl.ds(wb, NACC // 16)],
                            out_hbm.at[pl.ds(wb, NACC // 16)])

    return k


def _make_scatter_kernel(d):
    C = d // 32
    CHS = 4
    erows_sub = EROWS // 16
    nchunk = erows_sub // CHS

    @functools.partial(
        pl.kernel,
        mesh=_sc_mesh(),
        compiler_params=_SC_PARAMS,
        out_type=jax.ShapeDtypeStruct((C * N, 32), jnp.float32),
        scratch_types=[
            pltpu.VMEM((CHS, 128), jnp.int32),
            pltpu.VMEM((CHS, 128), jnp.int32),
            pltpu.VMEM((CHS, 128), jnp.int32),
            pltpu.VMEM((CHS * 128, 32), jnp.float32),
            pltpu.VMEM_SHARED((NACC, 32), jnp.float32),
            pltpu.SemaphoreType.DMA,
            pltpu.SemaphoreType.DMA,
        ],
    )
    def k(u_hbm, src_hbm, dst_hbm, zeros_hbm, out_hbm,
          src_v, dst_v, gidx_v, rows_v, acc_sh, gsem, ssem):
        cid = lax.axis_index("c")
        sid = lax.axis_index("s")

        for p in range(C // 2):
            chunk_id = p * 2 + cid
            zb = sid * (NACC // 16)
            pltpu.sync_copy(zeros_hbm.at[pl.ds(0, NACC // 16)],
                            acc_sh.at[pl.ds(zb, NACC // 16)])
            plsc.subcore_barrier()

            def chunk(i, _):
                rb = sid * erows_sub + i * CHS
                pltpu.sync_copy(src_hbm.at[pl.ds(rb, CHS)], src_v)
                pltpu.sync_copy(dst_hbm.at[pl.ds(rb, CHS)], dst_v)
                for r in range(CHS):
                    for m in range(8):
                        sv = src_v[r, pl.ds(m * 16, 16)]
                        gidx_v[r, pl.ds(m * 16, 16)] = sv * C + chunk_id
                gcps = [
                    pltpu.async_copy(u_hbm.at[gidx_v.at[r]],
                                     rows_v.at[pl.ds(r * 128, 128)], gsem)
                    for r in range(CHS)
                ]
                for cp in gcps:
                    cp.wait()
                scps = [
                    pltpu.async_copy(rows_v.at[pl.ds(r * 128, 128)],
                                     acc_sh.at[dst_v.at[r]], ssem,
                                     add=True)
                    for r in range(CHS)
                ]
                for cp in scps:
                    cp.wait()
                return 0

            lax.fori_loop(0, nchunk, chunk, 0)
            plsc.subcore_barrier()

            @pl.when(sid == 0)
            def _():
                pltpu.sync_copy(acc_sh.at[pl.ds(0, N)],
                                out_hbm.at[pl.ds(chunk_id * N, N)])

            if p + 1 < C // 2:
                plsc.subcore_barrier()

    return k


def _make_pool_kernel():
    RB = 16

    @functools.partial(
        pl.kernel,
        mesh=_sc_mesh(),
        compiler_params=_SC_PARAMS,
        out_type=(
            jax.ShapeDtypeStruct((B, 64), jnp.float32),
            jax.ShapeDtypeStruct((B, 64), jnp.float32),
            jax.ShapeDtypeStruct((B,), jnp.float32),
        ),
        scratch_types=[
            pltpu.VMEM((32,), jnp.int32),
            pltpu.VMEM((2, 64), jnp.float32),
            pltpu.VMEM((RB, 64), jnp.float32),
            pltpu.VMEM((16, 64), jnp.float32),
            pltpu.VMEM((16, 64), jnp.float32),
            pltpu.VMEM((16,), jnp.float32),
            pltpu.SemaphoreType.DMA,
        ],
    )
    def k(c3_hbm, st_hbm, starts_hbm, sums_hbm, maxs_hbm, cnts_hbm,
          se_v, st_v, row_v, outs_v, outm_v, outc_v, sem):
        cid = lax.axis_index("c")
        sid = lax.axis_index("s")
        g = sid * 2 + cid
        pltpu.sync_copy(st_hbm, st_v)
        pltpu.sync_copy(starts_hbm.at[pl.ds(g * 16, 32)], se_v)
        e0 = se_v[pl.ds(0, 16)]
        e1 = se_v[pl.ds(16, 16)]
        i16 = lax.iota(jnp.int32, 16)
        svec = [st_v[0, pl.ds(m * 16, 16)] for m in range(4)]
        tvec = [st_v[1, pl.ds(m * 16, 16)] for m in range(4)]

        def extract(j):
            a = jnp.where(i16 == j, e0, -2147483647)
            bb = jnp.where(i16 + 16 == j, e1, -2147483647)
            return jnp.max(jnp.maximum(a, bb))

        def seg(j, _):
            r0 = extract(j)
            r1 = extract(j + 1)
            cnt = r1 - r0
            nch = (cnt + RB - 1) // RB

            def chunk(i, carry):
                accs0, accs1, accs2, accs3, accm0, accm1, accm2, accm3 = carry
                intended = r0 + i * RB
                s2 = jnp.minimum(intended, N - RB)
                pltpu.sync_copy(c3_hbm.at[pl.ds(s2, RB)], row_v)
                accs = [accs0, accs1, accs2, accs3]
                accm = [accm0, accm1, accm2, accm3]
                for r in range(RB):
                    gr = s2 + r
                    val = jnp.logical_and(gr >= intended, gr < r1)
                    for m in range(4):
                        xv = row_v[r, pl.ds(m * 16, 16)]
                        y = jnp.maximum(xv * svec[m] + tvec[m], 0.0)
                        accs[m] = accs[m] + jnp.where(val, y, 0.0)
                        accm[m] = jnp.maximum(
                            accm[m], jnp.where(val, y, -jnp.inf))
                return tuple(accs) + tuple(accm)

            zero = jnp.zeros((16,), jnp.float32)
            ninf = jnp.full((16,), -jnp.inf, jnp.float32)
            res = lax.fori_loop(0, nch, chunk,
                                (zero, zero, zero, zero,
                                 ninf, ninf, ninf, ninf))
            for m in range(4):
                outs_v[j, pl.ds(m * 16, 16)] = res[m]
                outm_v[j, pl.ds(m * 16, 16)] = res[4 + m]
            cv = outc_v[pl.ds(0, 16)]
            outc_v[pl.ds(0, 16)] = jnp.where(
                i16 == j, cnt.astype(jnp.float32), cv)
            return 0

        lax.fori_loop(0, 16, seg, 0)
        pltpu.sync_copy(outs_v, sums_hbm.at[pl.ds(g * 16, 16)])
        pltpu.sync_copy(outm_v, maxs_hbm.at[pl.ds(g * 16, 16)])
        pltpu.sync_copy(outc_v, cnts_hbm.at[pl.ds(g * 16, 16)])

    return k


def _stage1_first(x, deg, W):
    dout = W.shape[1]

    def body(x_ref, deg_ref, w_ref, u_ref, dinv_ref):
        dv = lax.rsqrt(deg_ref[...] + 1.0)
        dinv_ref[...] = dv
        u_ref[...] = dv * jnp.dot(x_ref[...], w_ref[...],
                                  preferred_element_type=jnp.float32)

    return pl.pallas_call(
        body,
        grid=(NB,),
        in_specs=[
            pl.BlockSpec((BN_ROWS, x.shape[1]), lambda i: (i, 0)),
            pl.BlockSpec((BN_ROWS, 1), lambda i: (i, 0)),
            pl.BlockSpec(W.shape, lambda i: (0, 0)),
        ],
        out_specs=[
            pl.BlockSpec((BN_ROWS, dout), lambda i: (i, 0)),
            pl.BlockSpec((BN_ROWS, 1), lambda i: (i, 0)),
        ],
        out_shape=[
            jax.ShapeDtypeStruct((N, dout), jnp.float32),
            jax.ShapeDtypeStruct((N, 1), jnp.float32),
        ],
    )(x, deg, W)


def _stage1(c, sums, g, bt, W, dinv):
    din, dout = W.shape

    def body(c_ref, sums_ref, g_ref, bt_ref, w_ref, dinv_ref, u_ref):
        mean = sums_ref[0, :] * (1.0 / N)
        var = sums_ref[1, :] * (1.0 / N) - mean * mean
        s = g_ref[...] * lax.rsqrt(var + EPS)
        t = bt_ref[...] - mean * s
        a = jnp.maximum(c_ref[...] * s[None, :] + t[None, :], 0.0)
        u_ref[...] = dinv_ref[...] * jnp.dot(a, w_ref[...],
                                             preferred_element_type=jnp.float32)

    return pl.pallas_call(
        body,
        grid=(NB,),
        in_specs=[
            pl.BlockSpec((BN_ROWS, din), lambda i: (i, 0)),
            pl.BlockSpec((2, din), lambda i: (0, 0)),
            pl.BlockSpec((din,), lambda i: (0,)),
            pl.BlockSpec((din,), lambda i: (0,)),
            pl.BlockSpec((din, dout), lambda i: (0, 0)),
            pl.BlockSpec((BN_ROWS, 1), lambda i: (i, 0)),
        ],
        out_specs=pl.BlockSpec((BN_ROWS, dout), lambda i: (i, 0)),
        out_shape=jax.ShapeDtypeStruct((N, dout), jnp.float32),
    )(c, sums, g, bt, W, dinv)


def _stage2(acc, u, dinv, b, batch2d=None):
    d = u.shape[1]
    C = d // 32
    acc3 = acc.reshape(C, N, 32)
    with_starts = batch2d is not None

    def body(*refs):
        if with_starts:
            (acc_ref, u_ref, dinv_ref, b_ref, batch_ref,
             c_ref, sums_ref, starts_ref) = refs
        else:
            acc_ref, u_ref, dinv_ref, b_ref, c_ref, sums_ref = refs
        i = pl.program_id(0)
        acat = jnp.concatenate([acc_ref[ci] for ci in range(C)], axis=1)
        co = dinv_ref[...] * (acat + u_ref[...]) + b_ref[...][None, :]
        c_ref[...] = co
        part = jnp.concatenate(
            [jnp.sum(co, axis=0)[None, :],
             jnp.sum(co * co, axis=0)[None, :]], axis=0)

        @pl.when(i == 0)
        def _():
            sums_ref[...] = jnp.zeros_like(sums_ref)
            if with_starts:
                starts_ref[...] = jnp.zeros_like(starts_ref)

        sums_ref[...] += part

        if with_starts:
            ids = batch_ref[...]
            cmp = (ids < lax.broadcasted_iota(jnp.int32, (BN_ROWS, B), 1))
            starts_ref[...] += jnp.sum(
                cmp.astype(jnp.int32), axis=0)[None, :]

    in_specs = [
        pl.BlockSpec((C, BN_ROWS, 32), lambda i: (0, i, 0)),
        pl.BlockSpec((BN_ROWS, d), lambda i: (i, 0)),
        pl.BlockSpec((BN_ROWS, 1), lambda i: (i, 0)),
        pl.BlockSpec((d,), lambda i: (0,)),
    ]
    out_specs = [
        pl.BlockSpec((BN_ROWS, d), lambda i: (i, 0)),
        pl.BlockSpec((2, d), lambda i: (0, 0)),
    ]
    out_shape = [
        jax.ShapeDtypeStruct((N, d), jnp.float32),
        jax.ShapeDtypeStruct((2, d), jnp.float32),
    ]
    args = [acc3, u, dinv, b]
    if with_starts:
        in_specs.append(pl.BlockSpec((BN_ROWS, 1), lambda i: (i, 0)))
        out_specs.append(pl.BlockSpec((1, B), lambda i: (0, 0)))
        out_shape.append(jax.ShapeDtypeStruct((1, B), jnp.int32))
        args.append(batch2d)

    return pl.pallas_call(
        body,
        grid=(NB,),
        in_specs=in_specs,
        out_specs=out_specs,
        out_shape=out_shape,
    )(*args)


def _mlp(sums, maxs, cnts, fW1, fb1, fW2, fb2, fW3, fb3):
    def body(s_ref, m_ref, c_ref, w1_ref, b1_ref, w2_ref, b2_ref,
             w3_ref, b3_ref, o_ref):
        mean = s_ref[...] * (1.0 / jnp.maximum(c_ref[...], 1.0))
        z = jnp.dot(mean, w1_ref[0:64, :],
                    preferred_element_type=jnp.float32)
        z += jnp.dot(m_ref[...], w1_ref[64:128, :],
                     preferred_element_type=jnp.float32)
        z = jnp.maximum(z + b1_ref[...][None, :], 0.0)
        z = jnp.maximum(jnp.dot(z, w2_ref[...],
                                preferred_element_type=jnp.float32)
                        + b2_ref[...][None, :], 0.0)
        o_ref[...] = jnp.dot(z, w3_ref[...],
                             preferred_element_type=jnp.float32) \
            + b3_ref[...][None, :]

    return pl.pallas_call(
        body,
        out_shape=jax.ShapeDtypeStruct((B, 1), jnp.float32),
    )(sums, maxs, cnts, fW1, fb1, fW2, fb2, fW3, fb3)


_deg_k = _make_deg_kernel()
_scat = {64: _make_scatter_kernel(64), 128: _make_scatter_kernel(128)}
_pool_k = _make_pool_kernel()


def kernel(x, edge_index, batch, W1, b1, g1, bt1, W2, b2, g2, bt2,
           W3, b3, g3, bt3, fW1, fb1, fW2, fb2, fW3, fb3):
    src = edge_index[0]
    dst = edge_index[1]
    pad = EPAD - E
    srcp = jnp.concatenate(
        [src, jnp.zeros((pad,), jnp.int32)]).reshape(EROWS, 128)
    dstp = jnp.concatenate(
        [dst, jnp.full((pad,), N, jnp.int32)]).reshape(EROWS, 128)
    zeros1 = jnp.zeros((NACC // 16, 1), jnp.float32)
    zeros32 = jnp.zeros((NACC // 16, 32), jnp.float32)
    ones1 = jnp.ones((128, 1), jnp.float32)

    deg = _deg_k(dstp, ones1, zeros1)
    u1, dinv = _stage1_first(x, deg, W1)
    acc1 = _scat[64](u1.reshape(N * 2, 32), srcp, dstp, zeros32)
    c1, sums1 = _stage2(acc1, u1, dinv, b1)
    u2 = _stage1(c1, sums1, g1, bt1, W2, dinv)
    acc2 = _scat[128](u2.reshape(N * 4, 32), srcp, dstp, zeros32)
    c2, sums2 = _stage2(acc2, u2, dinv, b2)
    u3 = _stage1(c2, sums2, g2, bt2, W3, dinv)
    acc3 = _scat[64](u3.reshape(N * 2, 32), srcp, dstp, zeros32)
    c3, sums3, starts = _stage2(acc3, u3, dinv, b3,
                                batch2d=batch.reshape(N, 1))

    mean3 = sums3[0] * (1.0 / N)
    var3 = sums3[1] * (1.0 / N) - mean3 * mean3
    s3 = g3 * lax.rsqrt(var3 + EPS)
    t3 = bt3 - mean3 * s3
    st = jnp.stack([s3, t3])
    starts_ext = jnp.concatenate(
        [starts[0], jnp.full((32,), N, jnp.int32)])

    segsum, segmax, cnts = _pool_k(c3, st, starts_ext)
    return _mlp(segsum, segmax, cnts.reshape(B, 1),
                fW1, fb1, fW2, fb2, fW3, fb3)


# speedup vs baseline: 16.6276x; 1.1293x over previous
import functools

import jax
import jax.numpy as jnp
from jax import lax
from jax.experimental import pallas as pl
from jax.experimental.pallas import tpu as pltpu
from jax.experimental.pallas import tpu_sc as plsc

N = 50000
E = 800000
B = 512
EPS = 1e-5

BN_ROWS = 1000
NB = N // BN_ROWS
EROWS = 6272
EPAD = EROWS * 128
NACC = N + 48
CH = 8


def _sc_mesh():
    return plsc.VectorSubcoreMesh(core_axis_name="c", subcore_axis_name="s")


_SC_PARAMS = pltpu.CompilerParams(use_tc_tiling_on_sc=False,
                                  needs_layout_passes=False)


def _make_deg_kernel():
    nchunk = (EROWS // 16) // CH

    @functools.partial(
        pl.kernel,
        mesh=_sc_mesh(),
        compiler_params=_SC_PARAMS,
        out_type=jax.ShapeDtypeStruct((NACC, 1), jnp.float32),
        scratch_types=[
            pltpu.VMEM((CH, 128), jnp.int32),
            pltpu.VMEM((128, 1), jnp.float32),
            pltpu.VMEM_SHARED((NACC, 1), jnp.float32),
            pltpu.SemaphoreType.DMA,
        ],
    )
    def k(dst_hbm, ones_hbm, zeros_hbm, out_hbm, dst_v, ones_v, acc_sh, sem):
        cid = lax.axis_index("c")
        sid = lax.axis_index("s")
        pltpu.sync_copy(ones_hbm, ones_v)
        base = sid * (NACC // 16)
        pltpu.sync_copy(zeros_hbm.at[pl.ds(0, NACC // 16)],
                        acc_sh.at[pl.ds(base, NACC // 16)])
        plsc.subcore_barrier()

        def chunk(i, _):
            rb = sid * (EROWS // 16) + i * CH
            pltpu.sync_copy(dst_hbm.at[pl.ds(rb, CH)], dst_v)
            cps = [
                pltpu.async_copy(ones_v, acc_sh.at[dst_v.at[r]], sem,
                                 add=True)
                for r in range(CH)
            ]
            for cp in cps:
                cp.wait()
            return 0

        lax.fori_loop(0, nchunk, chunk, 0)
        plsc.subcore_barrier()

        @pl.when((sid // 8) == cid)
        def _():
            wb = sid * (NACC // 16)
            pltpu.sync_copy(acc_sh.at[pl.ds(wb, NACC // 16)],
                            out_hbm.at[pl.ds(wb, NACC // 16)])

    return k


def _make_scatter_kernel(d):
    C = d // 32
    CHS = 4
    erows_sub = EROWS // 16
    nchunk = erows_sub // CHS

    @functools.partial(
        pl.kernel,
        mesh=_sc_mesh(),
        compiler_params=_SC_PARAMS,
        out_type=jax.ShapeDtypeStruct((C * N, 32), jnp.float32),
        scratch_types=[
            pltpu.VMEM((CHS, 128), jnp.int32),
            pltpu.VMEM((CHS, 128), jnp.int32),
            pltpu.VMEM((CHS, 128), jnp.int32),
            pltpu.VMEM((CHS, 128), jnp.int32),
            pltpu.VMEM((CHS * 128, 32), jnp.float32),
            pltpu.VMEM_SHARED((NACC, 32), jnp.float32),
            pltpu.SemaphoreType.DMA,
            pltpu.SemaphoreType.DMA,
            pltpu.SemaphoreType.DMA,
            pltpu.SemaphoreType.DMA,
            pltpu.SemaphoreType.DMA,
        ],
    )
    def k(u_hbm, src_hbm, dst_hbm, zeros_hbm, out_hbm,
          src_v, dst_v0, dst_v1, gidx_v, rows_v, acc_sh,
          gsem0, gsem1, gsem2, gsem3, ssem):
        gsems = (gsem0, gsem1, gsem2, gsem3)
        dst_vs = (dst_v0, dst_v1)
        cid = lax.axis_index("c")
        sid = lax.axis_index("s")
        ebase = sid * erows_sub

        for p in range(C // 2):
            chunk_id = p * 2 + cid
            zb = sid * (NACC // 16)
            pltpu.sync_copy(zeros_hbm.at[pl.ds(0, NACC // 16)],
                            acc_sh.at[pl.ds(zb, NACC // 16)])
            plsc.subcore_barrier()

            def load_idx(x, b):
                rb = ebase + x * CHS
                pltpu.sync_copy(src_hbm.at[pl.ds(rb, CHS)], src_v)
                pltpu.sync_copy(dst_hbm.at[pl.ds(rb, CHS)], dst_vs[b])
                for r in range(CHS):
                    for m in range(8):
                        sv = src_v[r, pl.ds(m * 16, 16)]
                        gidx_v[r, pl.ds(m * 16, 16)] = sv * C + chunk_id

            load_idx(0, 0)

            def pair(i, _):
                for j in range(2):
                    x = 2 * i + j
                    gcps = [
                        pltpu.async_copy(u_hbm.at[gidx_v.at[r]],
                                         rows_v.at[pl.ds(r * 128, 128)],
                                         gsems[r])
                        for r in range(CHS)
                    ]
                    for cp in gcps:
                        cp.wait()
                    scps = [
                        pltpu.async_copy(rows_v.at[pl.ds(r * 128, 128)],
                                         acc_sh.at[dst_vs[j].at[r]], ssem,
                                         add=True)
                        for r in range(CHS)
                    ]
                    load_idx(x + 1, 1 - j)
                    for cp in scps:
                        cp.wait()
                return 0

            lax.fori_loop(0, nchunk // 2, pair, 0)
            plsc.subcore_barrier()

            @pl.when(sid == 0)
            def _():
                pltpu.sync_copy(acc_sh.at[pl.ds(0, N)],
                                out_hbm.at[pl.ds(chunk_id * N, N)])

            if p + 1 < C // 2:
                plsc.subcore_barrier()

    return k


def _make_pool_kernel():
    RB = 16

    @functools.partial(
        pl.kernel,
        mesh=_sc_mesh(),
        compiler_params=_SC_PARAMS,
        out_type=(
            jax.ShapeDtypeStruct((B, 64), jnp.float32),
            jax.ShapeDtypeStruct((B, 64), jnp.float32),
            jax.ShapeDtypeStruct((B,), jnp.float32),
        ),
        scratch_types=[
            pltpu.VMEM((32,), jnp.int32),
            pltpu.VMEM((2, 64), jnp.float32),
            pltpu.VMEM((RB, 64), jnp.float32),
            pltpu.VMEM((16, 64), jnp.float32),
            pltpu.VMEM((16, 64), jnp.float32),
            pltpu.VMEM((16,), jnp.float32),
            pltpu.SemaphoreType.DMA,
        ],
    )
    def k(c3_hbm, st_hbm, starts_hbm, sums_hbm, maxs_hbm, cnts_hbm,
          se_v, st_v, row_v, outs_v, outm_v, outc_v, sem):
        cid = lax.axis_index("c")
        sid = lax.axis_index("s")
        g = sid * 2 + cid
        pltpu.sync_copy(st_hbm, st_v)
        pltpu.sync_copy(starts_hbm.at[pl.ds(g * 16, 32)], se_v)
        e0 = se_v[pl.ds(0, 16)]
        e1 = se_v[pl.ds(16, 16)]
        i16 = lax.iota(jnp.int32, 16)
        svec = [st_v[0, pl.ds(m * 16, 16)] for m in range(4)]
        tvec = [st_v[1, pl.ds(m * 16, 16)] for m in range(4)]

        def extract(j):
            a = jnp.where(i16 == j, e0, -2147483647)
            bb = jnp.where(i16 + 16 == j, e1, -2147483647)
            return jnp.max(jnp.maximum(a, bb))

        def seg(j, _):
            r0 = extract(j)
            r1 = extract(j + 1)
            cnt = r1 - r0
            nch = (cnt + RB - 1) // RB

            def chunk(i, carry):
                accs0, accs1, accs2, accs3, accm0, accm1, accm2, accm3 = carry
                intended = r0 + i * RB
                s2 = jnp.minimum(intended, N - RB)
                pltpu.sync_copy(c3_hbm.at[pl.ds(s2, RB)], row_v)
                accs = [accs0, accs1, accs2, accs3]
                accm = [accm0, accm1, accm2, accm3]
                for r in range(RB):
                    gr = s2 + r
                    val = jnp.logical_and(gr >= intended, gr < r1)
                    for m in range(4):
                        xv = row_v[r, pl.ds(m * 16, 16)]
                        y = jnp.maximum(xv * svec[m] + tvec[m], 0.0)
                        accs[m] = accs[m] + jnp.where(val, y, 0.0)
                        accm[m] = jnp.maximum(
                            accm[m], jnp.where(val, y, -jnp.inf))
                return tuple(accs) + tuple(accm)

            zero = jnp.zeros((16,), jnp.float32)
            ninf = jnp.full((16,), -jnp.inf, jnp.float32)
            res = lax.fori_loop(0, nch, chunk,
                                (zero, zero, zero, zero,
                                 ninf, ninf, ninf, ninf))
            for m in range(4):
                outs_v[j, pl.ds(m * 16, 16)] = res[m]
                outm_v[j, pl.ds(m * 16, 16)] = res[4 + m]
            cv = outc_v[pl.ds(0, 16)]
            outc_v[pl.ds(0, 16)] = jnp.where(
                i16 == j, cnt.astype(jnp.float32), cv)
            return 0

        lax.fori_loop(0, 16, seg, 0)
        pltpu.sync_copy(outs_v, sums_hbm.at[pl.ds(g * 16, 16)])
        pltpu.sync_copy(outm_v, maxs_hbm.at[pl.ds(g * 16, 16)])
        pltpu.sync_copy(outc_v, cnts_hbm.at[pl.ds(g * 16, 16)])

    return k


def _stage1_first(x, deg, W):
    dout = W.shape[1]

    def body(x_ref, deg_ref, w_ref, u_ref, dinv_ref):
        dv = lax.rsqrt(deg_ref[...] + 1.0)
        dinv_ref[...] = dv
        u_ref[...] = dv * jnp.dot(x_ref[...], w_ref[...],
                                  preferred_element_type=jnp.float32)

    return pl.pallas_call(
        body,
        grid=(NB,),
        in_specs=[
            pl.BlockSpec((BN_ROWS, x.shape[1]), lambda i: (i, 0)),
            pl.BlockSpec((BN_ROWS, 1), lambda i: (i, 0)),
            pl.BlockSpec(W.shape, lambda i: (0, 0)),
        ],
        out_specs=[
            pl.BlockSpec((BN_ROWS, dout), lambda i: (i, 0)),
            pl.BlockSpec((BN_ROWS, 1), lambda i: (i, 0)),
        ],
        out_shape=[
            jax.ShapeDtypeStruct((N, dout), jnp.float32),
            jax.ShapeDtypeStruct((N, 1), jnp.float32),
        ],
    )(x, deg, W)


def _stage1(c, sums, g, bt, W, dinv):
    din, dout = W.shape

    def body(c_ref, sums_ref, g_ref, bt_ref, w_ref, dinv_ref, u_ref):
        mean = sums_ref[0, :] * (1.0 / N)
        var = sums_ref[1, :] * (1.0 / N) - mean * mean
        s = g_ref[...] * lax.rsqrt(var + EPS)
        t = bt_ref[...] - mean * s
        a = jnp.maximum(c_ref[...] * s[None, :] + t[None, :], 0.0)
        u_ref[...] = dinv_ref[...] * jnp.dot(a, w_ref[...],
                                             preferred_element_type=jnp.float32)

    return pl.pallas_call(
        body,
        grid=(NB,),
        in_specs=[
            pl.BlockSpec((BN_ROWS, din), lambda i: (i, 0)),
            pl.BlockSpec((2, din), lambda i: (0, 0)),
            pl.BlockSpec((din,), lambda i: (0,)),
            pl.BlockSpec((din,), lambda i: (0,)),
            pl.BlockSpec((din, dout), lambda i: (0, 0)),
            pl.BlockSpec((BN_ROWS, 1), lambda i: (i, 0)),
        ],
        out_specs=pl.BlockSpec((BN_ROWS, dout), lambda i: (i, 0)),
        out_shape=jax.ShapeDtypeStruct((N, dout), jnp.float32),
    )(c, sums, g, bt, W, dinv)


def _stage2(acc, u, dinv, b, batch2d=None):
    d = u.shape[1]
    C = d // 32
    acc3 = acc.reshape(C, N, 32)
    with_starts = batch2d is not None

    def body(*refs):
        if with_starts:
            (acc_ref, u_ref, dinv_ref, b_ref, batch_ref,
             c_ref, sums_ref, starts_ref) = refs
        else:
            acc_ref, u_ref, dinv_ref, b_ref, c_ref, sums_ref = refs
        i = pl.program_id(0)
        acat = jnp.concatenate([acc_ref[ci] for ci in range(C)], axis=1)
        co = dinv_ref[...] * (acat + u_ref[...]) + b_ref[...][None, :]
        c_ref[...] = co
        part = jnp.concatenate(
            [jnp.sum(co, axis=0)[None, :],
             jnp.sum(co * co, axis=0)[None, :]], axis=0)

        @pl.when(i == 0)
        def _():
            sums_ref[...] = jnp.zeros_like(sums_ref)
            if with_starts:
                starts_ref[...] = jnp.zeros_like(starts_ref)

        sums_ref[...] += part

        if with_starts:
            ids = batch_ref[...]
            cmp = (ids < lax.broadcasted_iota(jnp.int32, (BN_ROWS, B), 1))
            starts_ref[...] += jnp.sum(
                cmp.astype(jnp.int32), axis=0)[None, :]

    in_specs = [
        pl.BlockSpec((C, BN_ROWS, 32), lambda i: (0, i, 0)),
        pl.BlockSpec((BN_ROWS, d), lambda i: (i, 0)),
        pl.BlockSpec((BN_ROWS, 1), lambda i: (i, 0)),
        pl.BlockSpec((d,), lambda i: (0,)),
    ]
    out_specs = [
        pl.BlockSpec((BN_ROWS, d), lambda i: (i, 0)),
        pl.BlockSpec((2, d), lambda i: (0, 0)),
    ]
    out_shape = [
        jax.ShapeDtypeStruct((N, d), jnp.float32),
        jax.ShapeDtypeStruct((2, d), jnp.float32),
    ]
    args = [acc3, u, dinv, b]
    if with_starts:
        in_specs.append(pl.BlockSpec((BN_ROWS, 1), lambda i: (i, 0)))
        out_specs.append(pl.BlockSpec((1, B), lambda i: (0, 0)))
        out_shape.append(jax.ShapeDtypeStruct((1, B), jnp.int32))
        args.append(batch2d)

    return pl.pallas_call(
        body,
        grid=(NB,),
        in_specs=in_specs,
        out_specs=out_specs,
        out_shape=out_shape,
    )(*args)


def _mlp(sums, maxs, cnts, fW1, fb1, fW2, fb2, fW3, fb3):
    def body(s_ref, m_ref, c_ref, w1_ref, b1_ref, w2_ref, b2_ref,
             w3_ref, b3_ref, o_ref):
        mean = s_ref[...] * (1.0 / jnp.maximum(c_ref[...], 1.0))
        z = jnp.dot(mean, w1_ref[0:64, :],
                    preferred_element_type=jnp.float32)
        z += jnp.dot(m_ref[...], w1_ref[64:128, :],
                     preferred_element_type=jnp.float32)
        z = jnp.maximum(z + b1_ref[...][None, :], 0.0)
        z = jnp.maximum(jnp.dot(z, w2_ref[...],
                                preferred_element_type=jnp.float32)
                        + b2_ref[...][None, :], 0.0)
        o_ref[...] = jnp.dot(z, w3_ref[...],
                             preferred_element_type=jnp.float32) \
            + b3_ref[...][None, :]

    return pl.pallas_call(
        body,
        out_shape=jax.ShapeDtypeStruct((B, 1), jnp.float32),
    )(sums, maxs, cnts, fW1, fb1, fW2, fb2, fW3, fb3)


_deg_k = _make_deg_kernel()
_scat = {64: _make_scatter_kernel(64), 128: _make_scatter_kernel(128)}
_pool_k = _make_pool_kernel()


def kernel(x, edge_index, batch, W1, b1, g1, bt1, W2, b2, g2, bt2,
           W3, b3, g3, bt3, fW1, fb1, fW2, fb2, fW3, fb3):
    src = edge_index[0]
    dst = edge_index[1]
    pad = EPAD + CH * 16 * 128 - E
    srcp = jnp.concatenate(
        [src, jnp.zeros((pad,), jnp.int32)]).reshape(-1, 128)
    dstp = jnp.concatenate(
        [dst, jnp.full((pad,), N, jnp.int32)]).reshape(-1, 128)
    zeros1 = jnp.zeros((NACC // 16, 1), jnp.float32)
    zeros32 = jnp.zeros((NACC // 16, 32), jnp.float32)
    ones1 = jnp.ones((128, 1), jnp.float32)

    deg = _deg_k(dstp, ones1, zeros1)
    u1, dinv = _stage1_first(x, deg, W1)
    acc1 = _scat[64](u1.reshape(N * 2, 32), srcp, dstp, zeros32)
    c1, sums1 = _stage2(acc1, u1, dinv, b1)
    u2 = _stage1(c1, sums1, g1, bt1, W2, dinv)
    acc2 = _scat[128](u2.reshape(N * 4, 32), srcp, dstp, zeros32)
    c2, sums2 = _stage2(acc2, u2, dinv, b2)
    u3 = _stage1(c2, sums2, g2, bt2, W3, dinv)
    acc3 = _scat[64](u3.reshape(N * 2, 32), srcp, dstp, zeros32)
    c3, sums3, starts = _stage2(acc3, u3, dinv, b3,
                                batch2d=batch.reshape(N, 1))

    mean3 = sums3[0] * (1.0 / N)
    var3 = sums3[1] * (1.0 / N) - mean3 * mean3
    s3 = g3 * lax.rsqrt(var3 + EPS)
    t3 = bt3 - mean3 * s3
    st = jnp.stack([s3, t3])
    starts_ext = jnp.concatenate(
        [starts[0], jnp.full((32,), N, jnp.int32)])

    segsum, segmax, cnts = _pool_k(c3, st, starts_ext)
    return _mlp(segsum, segmax, cnts.reshape(B, 1),
                fW1, fb1, fW2, fb2, fW3, fb3)


# speedup vs baseline: 17.4269x; 1.0481x over previous
import functools

import jax
import jax.numpy as jnp
from jax import lax
from jax.experimental import pallas as pl
from jax.experimental.pallas import tpu as pltpu
from jax.experimental.pallas import tpu_sc as plsc

N = 50000
E = 800000
B = 512
EPS = 1e-5

BN_ROWS = 2000
NB = N // BN_ROWS
EROWS = 6272
EPAD = EROWS * 128
NACC = N + 48
CH = 8


def _sc_mesh():
    return plsc.VectorSubcoreMesh(core_axis_name="c", subcore_axis_name="s")


_SC_PARAMS = pltpu.CompilerParams(use_tc_tiling_on_sc=False,
                                  needs_layout_passes=False)


def _make_deg_kernel():
    nchunk = (EROWS // 16) // CH

    @functools.partial(
        pl.kernel,
        mesh=_sc_mesh(),
        compiler_params=_SC_PARAMS,
        out_type=jax.ShapeDtypeStruct((NACC, 1), jnp.float32),
        scratch_types=[
            pltpu.VMEM((CH, 128), jnp.int32),
            pltpu.VMEM((128, 1), jnp.float32),
            pltpu.VMEM_SHARED((NACC, 1), jnp.float32),
            pltpu.SemaphoreType.DMA,
        ],
    )
    def k(dst_hbm, ones_hbm, zeros_hbm, out_hbm, dst_v, ones_v, acc_sh, sem):
        cid = lax.axis_index("c")
        sid = lax.axis_index("s")
        pltpu.sync_copy(ones_hbm, ones_v)
        base = sid * (NACC // 16)
        pltpu.sync_copy(zeros_hbm.at[pl.ds(0, NACC // 16)],
                        acc_sh.at[pl.ds(base, NACC // 16)])
        plsc.subcore_barrier()

        def chunk(i, _):
            rb = sid * (EROWS // 16) + i * CH
            pltpu.sync_copy(dst_hbm.at[pl.ds(rb, CH)], dst_v)
            cps = [
                pltpu.async_copy(ones_v, acc_sh.at[dst_v.at[r]], sem,
                                 add=True)
                for r in range(CH)
            ]
            for cp in cps:
                cp.wait()
            return 0

        lax.fori_loop(0, nchunk, chunk, 0)
        plsc.subcore_barrier()

        @pl.when((sid // 8) == cid)
        def _():
            wb = sid * (NACC // 16)
            pltpu.sync_copy(acc_sh.at[pl.ds(wb, NACC // 16)],
                            out_hbm.at[pl.ds(wb, NACC // 16)])

    return k


def _make_scatter_kernel(d):
    C = d // 32
    CHS = 4
    erows_sub = EROWS // 16
    nchunk = erows_sub // CHS

    @functools.partial(
        pl.kernel,
        mesh=_sc_mesh(),
        compiler_params=_SC_PARAMS,
        out_type=jax.ShapeDtypeStruct((C * N, 32), jnp.float32),
        scratch_types=[
            pltpu.VMEM((CHS, 128), jnp.int32),
            pltpu.VMEM((CHS, 128), jnp.int32),
            pltpu.VMEM((CHS, 128), jnp.int32),
            pltpu.VMEM((CHS, 128), jnp.int32),
            pltpu.VMEM((CHS * 128, 32), jnp.float32),
            pltpu.VMEM_SHARED((NACC, 32), jnp.float32),
            pltpu.SemaphoreType.DMA,
            pltpu.SemaphoreType.DMA,
            pltpu.SemaphoreType.DMA,
            pltpu.SemaphoreType.DMA,
            pltpu.SemaphoreType.DMA,
        ],
    )
    def k(u_hbm, src_hbm, dst_hbm, zeros_hbm, out_hbm,
          src_v, dst_v0, dst_v1, gidx_v, rows_v, acc_sh,
          gsem0, gsem1, gsem2, gsem3, ssem):
        gsems = (gsem0, gsem1, gsem2, gsem3)
        dst_vs = (dst_v0, dst_v1)
        cid = lax.axis_index("c")
        sid = lax.axis_index("s")
        ebase = sid * erows_sub

        for p in range(C // 2):
            chunk_id = p * 2 + cid
            zb = sid * (NACC // 16)
            pltpu.sync_copy(zeros_hbm.at[pl.ds(0, NACC // 16)],
                            acc_sh.at[pl.ds(zb, NACC // 16)])
            plsc.subcore_barrier()

            def load_idx(x, b):
                rb = ebase + x * CHS
                pltpu.sync_copy(src_hbm.at[pl.ds(rb, CHS)], src_v)
                pltpu.sync_copy(dst_hbm.at[pl.ds(rb, CHS)], dst_vs[b])
                for r in range(CHS):
                    for m in range(8):
                        sv = src_v[r, pl.ds(m * 16, 16)]
                        gidx_v[r, pl.ds(m * 16, 16)] = sv * C + chunk_id

            load_idx(0, 0)

            def pair(i, _):
                for j in range(2):
                    x = 2 * i + j
                    gcps = [
                        pltpu.async_copy(u_hbm.at[gidx_v.at[r]],
                                         rows_v.at[pl.ds(r * 128, 128)],
                                         gsems[r])
                        for r in range(CHS)
                    ]
                    for cp in gcps:
                        cp.wait()
                    scps = [
                        pltpu.async_copy(rows_v.at[pl.ds(r * 128, 128)],
                                         acc_sh.at[dst_vs[j].at[r]], ssem,
                                         add=True)
                        for r in range(CHS)
                    ]
                    load_idx(x + 1, 1 - j)
                    for cp in scps:
                        cp.wait()
                return 0

            lax.fori_loop(0, nchunk // 2, pair, 0)
            plsc.subcore_barrier()

            @pl.when(sid == 0)
            def _():
                pltpu.sync_copy(acc_sh.at[pl.ds(0, N)],
                                out_hbm.at[pl.ds(chunk_id * N, N)])

            if p + 1 < C // 2:
                plsc.subcore_barrier()

    return k


def _make_pool_kernel():
    RB = 16

    @functools.partial(
        pl.kernel,
        mesh=_sc_mesh(),
        compiler_params=_SC_PARAMS,
        out_type=(
            jax.ShapeDtypeStruct((B, 64), jnp.float32),
            jax.ShapeDtypeStruct((B, 64), jnp.float32),
            jax.ShapeDtypeStruct((B,), jnp.float32),
        ),
        scratch_types=[
            pltpu.VMEM((32,), jnp.int32),
            pltpu.VMEM((2, 64), jnp.float32),
            pltpu.VMEM((RB, 64), jnp.float32),
            pltpu.VMEM((16, 64), jnp.float32),
            pltpu.VMEM((16, 64), jnp.float32),
            pltpu.VMEM((16,), jnp.float32),
            pltpu.SemaphoreType.DMA,
        ],
    )
    def k(c3_hbm, st_hbm, starts_hbm, sums_hbm, maxs_hbm, cnts_hbm,
          se_v, st_v, row_v, outs_v, outm_v, outc_v, sem):
        cid = lax.axis_index("c")
        sid = lax.axis_index("s")
        g = sid * 2 + cid
        pltpu.sync_copy(st_hbm, st_v)
        pltpu.sync_copy(starts_hbm.at[pl.ds(g * 16, 32)], se_v)
        e0 = se_v[pl.ds(0, 16)]
        e1 = se_v[pl.ds(16, 16)]
        i16 = lax.iota(jnp.int32, 16)
        svec = [st_v[0, pl.ds(m * 16, 16)] for m in range(4)]
        tvec = [st_v[1, pl.ds(m * 16, 16)] for m in range(4)]

        def extract(j):
            a = jnp.where(i16 == j, e0, -2147483647)
            bb = jnp.where(i16 + 16 == j, e1, -2147483647)
            return jnp.max(jnp.maximum(a, bb))

        def seg(j, _):
            r0 = extract(j)
            r1 = extract(j + 1)
            cnt = r1 - r0
            nch = (cnt + RB - 1) // RB

            def chunk(i, carry):
                accs0, accs1, accs2, accs3, accm0, accm1, accm2, accm3 = carry
                intended = r0 + i * RB
                s2 = jnp.minimum(intended, N - RB)
                pltpu.sync_copy(c3_hbm.at[pl.ds(s2, RB)], row_v)
                accs = [accs0, accs1, accs2, accs3]
                accm = [accm0, accm1, accm2, accm3]
                for r in range(RB):
                    gr = s2 + r
                    val = jnp.logical_and(gr >= intended, gr < r1)
                    for m in range(4):
                        xv = row_v[r, pl.ds(m * 16, 16)]
                        y = jnp.maximum(xv * svec[m] + tvec[m], 0.0)
                        accs[m] = accs[m] + jnp.where(val, y, 0.0)
                        accm[m] = jnp.maximum(
                            accm[m], jnp.where(val, y, -jnp.inf))
                return tuple(accs) + tuple(accm)

            zero = jnp.zeros((16,), jnp.float32)
            ninf = jnp.full((16,), -jnp.inf, jnp.float32)
            res = lax.fori_loop(0, nch, chunk,
                                (zero, zero, zero, zero,
                                 ninf, ninf, ninf, ninf))
            for m in range(4):
                outs_v[j, pl.ds(m * 16, 16)] = res[m]
                outm_v[j, pl.ds(m * 16, 16)] = res[4 + m]
            cv = outc_v[pl.ds(0, 16)]
            outc_v[pl.ds(0, 16)] = jnp.where(
                i16 == j, cnt.astype(jnp.float32), cv)
            return 0

        lax.fori_loop(0, 16, seg, 0)
        pltpu.sync_copy(outs_v, sums_hbm.at[pl.ds(g * 16, 16)])
        pltpu.sync_copy(outm_v, maxs_hbm.at[pl.ds(g * 16, 16)])
        pltpu.sync_copy(outc_v, cnts_hbm.at[pl.ds(g * 16, 16)])

    return k


def _stage1_first(x, deg, W):
    dout = W.shape[1]

    def body(x_ref, deg_ref, w_ref, u_ref, dinv_ref):
        dv = lax.rsqrt(deg_ref[...] + 1.0)
        dinv_ref[...] = dv
        u_ref[...] = dv * jnp.dot(x_ref[...], w_ref[...],
                                  preferred_element_type=jnp.float32)

    return pl.pallas_call(
        body,
        grid=(NB,),
        in_specs=[
            pl.BlockSpec((BN_ROWS, x.shape[1]), lambda i: (i, 0)),
            pl.BlockSpec((BN_ROWS, 1), lambda i: (i, 0)),
            pl.BlockSpec(W.shape, lambda i: (0, 0)),
        ],
        out_specs=[
            pl.BlockSpec((BN_ROWS, dout), lambda i: (i, 0)),
            pl.BlockSpec((BN_ROWS, 1), lambda i: (i, 0)),
        ],
        out_shape=[
            jax.ShapeDtypeStruct((N, dout), jnp.float32),
            jax.ShapeDtypeStruct((N, 1), jnp.float32),
        ],
    )(x, deg, W)


def _stage1(c, sums, g, bt, W, dinv):
    din, dout = W.shape

    def body(c_ref, sums_ref, g_ref, bt_ref, w_ref, dinv_ref, u_ref):
        mean = sums_ref[0, :] * (1.0 / N)
        var = sums_ref[1, :] * (1.0 / N) - mean * mean
        s = g_ref[...] * lax.rsqrt(var + EPS)
        t = bt_ref[...] - mean * s
        a = jnp.maximum(c_ref[...] * s[None, :] + t[None, :], 0.0)
        u_ref[...] = dinv_ref[...] * jnp.dot(a, w_ref[...],
                                             preferred_element_type=jnp.float32)

    return pl.pallas_call(
        body,
        grid=(NB,),
        in_specs=[
            pl.BlockSpec((BN_ROWS, din), lambda i: (i, 0)),
            pl.BlockSpec((2, din), lambda i: (0, 0)),
            pl.BlockSpec((din,), lambda i: (0,)),
            pl.BlockSpec((din,), lambda i: (0,)),
            pl.BlockSpec((din, dout), lambda i: (0, 0)),
            pl.BlockSpec((BN_ROWS, 1), lambda i: (i, 0)),
        ],
        out_specs=pl.BlockSpec((BN_ROWS, dout), lambda i: (i, 0)),
        out_shape=jax.ShapeDtypeStruct((N, dout), jnp.float32),
    )(c, sums, g, bt, W, dinv)


def _stage2(acc, u, dinv, b, batch2d=None):
    d = u.shape[1]
    C = d // 32
    acc3 = acc.reshape(C, N, 32)
    with_starts = batch2d is not None

    def body(*refs):
        if with_starts:
            (acc_ref, u_ref, dinv_ref, b_ref, batch_ref,
             c_ref, sums_ref, starts_ref) = refs
        else:
            acc_ref, u_ref, dinv_ref, b_ref, c_ref, sums_ref = refs
        i = pl.program_id(0)
        acat = jnp.concatenate([acc_ref[ci] for ci in range(C)], axis=1)
        co = dinv_ref[...] * (acat + u_ref[...]) + b_ref[...][None, :]
        c_ref[...] = co
        part = jnp.concatenate(
            [jnp.sum(co, axis=0)[None, :],
             jnp.sum(co * co, axis=0)[None, :]], axis=0)

        @pl.when(i == 0)
        def _():
            sums_ref[...] = jnp.zeros_like(sums_ref)
            if with_starts:
                starts_ref[...] = jnp.zeros_like(starts_ref)

        sums_ref[...] += part

        if with_starts:
            ids = batch_ref[...]
            cmp = (ids < lax.broadcasted_iota(jnp.int32, (BN_ROWS, B), 1))
            starts_ref[...] += jnp.sum(
                cmp.astype(jnp.int32), axis=0)[None, :]

    in_specs = [
        pl.BlockSpec((C, BN_ROWS, 32), lambda i: (0, i, 0)),
        pl.BlockSpec((BN_ROWS, d), lambda i: (i, 0)),
        pl.BlockSpec((BN_ROWS, 1), lambda i: (i, 0)),
        pl.BlockSpec((d,), lambda i: (0,)),
    ]
    out_specs = [
        pl.BlockSpec((BN_ROWS, d), lambda i: (i, 0)),
        pl.BlockSpec((2, d), lambda i: (0, 0)),
    ]
    out_shape = [
        jax.ShapeDtypeStruct((N, d), jnp.float32),
        jax.ShapeDtypeStruct((2, d), jnp.float32),
    ]
    args = [acc3, u, dinv, b]
    if with_starts:
        in_specs.append(pl.BlockSpec((BN_ROWS, 1), lambda i: (i, 0)))
        out_specs.append(pl.BlockSpec((1, B), lambda i: (0, 0)))
        out_shape.append(jax.ShapeDtypeStruct((1, B), jnp.int32))
        args.append(batch2d)

    return pl.pallas_call(
        body,
        grid=(NB,),
        in_specs=in_specs,
        out_specs=out_specs,
        out_shape=out_shape,
    )(*args)


def _mlp(sums, maxs, cnts, fW1, fb1, fW2, fb2, fW3, fb3):
    def body(s_ref, m_ref, c_ref, w1_ref, b1_ref, w2_ref, b2_ref,
             w3_ref, b3_ref, o_ref):
        mean = s_ref[...] * (1.0 / jnp.maximum(c_ref[...], 1.0))
        z = jnp.dot(mean, w1_ref[0:64, :],
                    preferred_element_type=jnp.float32)
        z += jnp.dot(m_ref[...], w1_ref[64:128, :],
                     preferred_element_type=jnp.float32)
        z = jnp.maximum(z + b1_ref[...][None, :], 0.0)
        z = jnp.maximum(jnp.dot(z, w2_ref[...],
                                preferred_element_type=jnp.float32)
                        + b2_ref[...][None, :], 0.0)
        o_ref[...] = jnp.dot(z, w3_ref[...],
                             preferred_element_type=jnp.float32) \
            + b3_ref[...][None, :]

    return pl.pallas_call(
        body,
        out_shape=jax.ShapeDtypeStruct((B, 1), jnp.float32),
    )(sums, maxs, cnts, fW1, fb1, fW2, fb2, fW3, fb3)


_deg_k = _make_deg_kernel()
_scat = {64: _make_scatter_kernel(64), 128: _make_scatter_kernel(128)}
_pool_k = _make_pool_kernel()


def kernel(x, edge_index, batch, W1, b1, g1, bt1, W2, b2, g2, bt2,
           W3, b3, g3, bt3, fW1, fb1, fW2, fb2, fW3, fb3):
    src = edge_index[0]
    dst = edge_index[1]
    pad = EPAD + CH * 16 * 128 - E
    srcp = jnp.concatenate(
        [src, jnp.zeros((pad,), jnp.int32)]).reshape(-1, 128)
    dstp = jnp.concatenate(
        [dst, jnp.full((pad,), N, jnp.int32)]).reshape(-1, 128)
    zeros1 = jnp.zeros((NACC // 16, 1), jnp.float32)
    zeros32 = jnp.zeros((NACC // 16, 32), jnp.float32)
    ones1 = jnp.ones((128, 1), jnp.float32)

    deg = _deg_k(dstp, ones1, zeros1)
    u1, dinv = _stage1_first(x, deg, W1)
    acc1 = _scat[64](u1.reshape(N * 2, 32), srcp, dstp, zeros32)
    c1, sums1 = _stage2(acc1, u1, dinv, b1)
    u2 = _stage1(c1, sums1, g1, bt1, W2, dinv)
    acc2 = _scat[128](u2.reshape(N * 4, 32), srcp, dstp, zeros32)
    c2, sums2 = _stage2(acc2, u2, dinv, b2)
    u3 = _stage1(c2, sums2, g2, bt2, W3, dinv)
    acc3 = _scat[64](u3.reshape(N * 2, 32), srcp, dstp, zeros32)
    c3, sums3, starts = _stage2(acc3, u3, dinv, b3,
                                batch2d=batch.reshape(N, 1))

    mean3 = sums3[0] * (1.0 / N)
    var3 = sums3[1] * (1.0 / N) - mean3 * mean3
    s3 = g3 * lax.rsqrt(var3 + EPS)
    t3 = bt3 - mean3 * s3
    st = jnp.stack([s3, t3])
    starts_ext = jnp.concatenate(
        [starts[0], jnp.full((32,), N, jnp.int32)])

    segsum, segmax, cnts = _pool_k(c3, st, starts_ext)
    return _mlp(segsum, segmax, cnts.reshape(B, 1),
                fW1, fb1, fW2, fb2, fW3, fb3)
